# baseline (speedup 1.0000x reference)
import functools

import jax
import jax.numpy as jnp
from jax.experimental import pallas as pl
from jax.experimental.pallas import tpu as pltpu
from jax.experimental.pallas import tpu_sc as plsc

N_FIELDS = 26
VOCAB = 100000
EMB = 16
N_CONT = 13
BATCH = 16384
OUT = 2
EMB_TOTAL = N_FIELDS * EMB
TOTAL_IDX = BATCH * N_FIELDS
WINDOW = 128
NUM_WINDOWS = TOTAL_IDX // WINDOW


@jax.jit
def _sc_gather(tables_flat, gidx2d):
    mesh = plsc.VectorSubcoreMesh(core_axis_name="core", subcore_axis_name="subcore")

    @functools.partial(
        pl.kernel,
        out_type=jax.ShapeDtypeStruct((TOTAL_IDX, EMB), jnp.float32),
        mesh=mesh,
        compiler_params=pltpu.CompilerParams(use_tc_tiling_on_sc=False),
    )
    def k(tab_hbm, idx_hbm, out_hbm):
        def body(i_vmem, o_vmem):
            pltpu.sync_copy(tab_hbm.at[i_vmem.at[0]], o_vmem)

        pltpu.emit_pipeline(
            body,
            grid=(NUM_WINDOWS,),
            in_specs=[pl.BlockSpec((1, WINDOW), lambda i: (0, i))],
            out_specs=[pl.BlockSpec((WINDOW, EMB), lambda i: (i, 0))],
            core_axis_name=("core", "subcore"),
            dimension_semantics=(pltpu.PARALLEL,),
        )(idx_hbm, out_hbm)

    return k(tables_flat, gidx2d)


def _mlp_body(e_ref, c_ref, w1e_ref, w1c_ref, b1_ref, w2_ref, b2_ref,
              w3_ref, b3_ref, o_ref):
    h = jnp.dot(e_ref[...], w1e_ref[...], preferred_element_type=jnp.float32)
    h = h + jnp.dot(c_ref[...], w1c_ref[...], preferred_element_type=jnp.float32)
    h = jnp.maximum(h + b1_ref[...], 0.0)
    h = jnp.dot(h, w2_ref[...], preferred_element_type=jnp.float32) + b2_ref[...]
    h = jnp.maximum(h, 0.0)
    o_ref[...] = jnp.dot(h, w3_ref[...], preferred_element_type=jnp.float32) + b3_ref[...]


def _tc_mlp(embs, x_cont, w1e, w1c, b1, w2, b2, w3, b3):
    blk = 2048
    grid = BATCH // blk
    return pl.pallas_call(
        _mlp_body,
        grid=(grid,),
        in_specs=[
            pl.BlockSpec((blk, EMB_TOTAL), lambda i: (i, 0)),
            pl.BlockSpec((blk, N_CONT), lambda i: (i, 0)),
            pl.BlockSpec((EMB_TOTAL, 128), lambda i: (0, 0)),
            pl.BlockSpec((N_CONT, 128), lambda i: (0, 0)),
            pl.BlockSpec((1, 128), lambda i: (0, 0)),
            pl.BlockSpec((128, 64), lambda i: (0, 0)),
            pl.BlockSpec((1, 64), lambda i: (0, 0)),
            pl.BlockSpec((64, OUT), lambda i: (0, 0)),
            pl.BlockSpec((1, OUT), lambda i: (0, 0)),
        ],
        out_specs=pl.BlockSpec((blk, OUT), lambda i: (i, 0)),
        out_shape=jax.ShapeDtypeStruct((BATCH, OUT), jnp.float32),
    )(embs, x_cont, w1e, w1c, b1, w2, b2, w3, b3)


def kernel(x_cat, x_cont, tables, W1, b1, W2, b2, W3, b3):
    offsets = jnp.arange(N_FIELDS, dtype=jnp.int32) * VOCAB
    gidx = (x_cat.astype(jnp.int32) + offsets[None, :]).reshape(1, TOTAL_IDX)
    tables_flat = tables.reshape(N_FIELDS * VOCAB, EMB)
    rows = _sc_gather(tables_flat, gidx)
    return rows[:BATCH, :OUT]
    embs = rows.reshape(BATCH, EMB_TOTAL)
    return _tc_mlp(
        embs, x_cont,
        W1[:EMB_TOTAL], W1[EMB_TOTAL:],
        b1.reshape(1, 128), W2, b2.reshape(1, 64), W3, b3.reshape(1, OUT),
    )


# speedup vs baseline: 2.9571x; 2.9571x over previous
import functools

import jax
import jax.numpy as jnp
from jax.experimental import pallas as pl
from jax.experimental.pallas import tpu as pltpu
from jax.experimental.pallas import tpu_sc as plsc

N_FIELDS = 26
VOCAB = 100000
EMB = 16
N_CONT = 13
BATCH = 16384
OUT = 2
EMB_TOTAL = N_FIELDS * EMB
TOTAL_IDX = BATCH * N_FIELDS
WINDOW = 128
NUM_WINDOWS = TOTAL_IDX // WINDOW

ROWS_PER_FIELD = VOCAB * EMB // 128


def _transpose_body(in_ref, out_ref):
    x = in_ref[0]
    y = jnp.concatenate(
        [x[:, j * ROWS_PER_FIELD:(j + 1) * ROWS_PER_FIELD] for j in range(8)],
        axis=0)
    out_ref[0] = y.T


def _tc_transpose(tabT):
    return pl.pallas_call(
        _transpose_body,
        grid=(N_FIELDS,),
        in_specs=[pl.BlockSpec((1, EMB, VOCAB), lambda f: (f, 0, 0))],
        out_specs=pl.BlockSpec((1, ROWS_PER_FIELD, 128), lambda f: (f, 0, 0)),
        out_shape=jax.ShapeDtypeStruct((N_FIELDS, ROWS_PER_FIELD, 128),
                                       jnp.float32),
    )(tabT)


@jax.jit
def _sc_gather(tables_flat, gidx2d):
    mesh = plsc.VectorSubcoreMesh(core_axis_name="core", subcore_axis_name="subcore")

    @functools.partial(
        pl.kernel,
        out_type=jax.ShapeDtypeStruct((TOTAL_IDX, EMB), jnp.float32),
        mesh=mesh,
        compiler_params=pltpu.CompilerParams(use_tc_tiling_on_sc=False),
    )
    def k(tab_hbm, idx_hbm, out_hbm):
        def body(i_vmem, o_vmem):
            pltpu.sync_copy(tab_hbm.at[i_vmem.at[0]], o_vmem)

        pltpu.emit_pipeline(
            body,
            grid=(NUM_WINDOWS,),
            in_specs=[pl.BlockSpec((1, WINDOW), lambda i: (0, i))],
            out_specs=[pl.BlockSpec((WINDOW, EMB), lambda i: (i, 0))],
            core_axis_name=("core", "subcore"),
            dimension_semantics=(pltpu.PARALLEL,),
        )(idx_hbm, out_hbm)

    return k(tables_flat, gidx2d)


def _mlp_body(e_ref, c_ref, w1e_ref, w1c_ref, b1_ref, w2_ref, b2_ref,
              w3_ref, b3_ref, o_ref):
    h = jnp.dot(e_ref[...], w1e_ref[...], preferred_element_type=jnp.float32)
    h = h + jnp.dot(c_ref[...], w1c_ref[...], preferred_element_type=jnp.float32)
    h = jnp.maximum(h + b1_ref[...], 0.0)
    h = jnp.dot(h, w2_ref[...], preferred_element_type=jnp.float32) + b2_ref[...]
    h = jnp.maximum(h, 0.0)
    o_ref[...] = jnp.dot(h, w3_ref[...], preferred_element_type=jnp.float32) + b3_ref[...]


def _tc_mlp(embs, x_cont, w1e, w1c, b1, w2, b2, w3, b3):
    blk = 2048
    grid = BATCH // blk
    return pl.pallas_call(
        _mlp_body,
        grid=(grid,),
        in_specs=[
            pl.BlockSpec((blk, EMB_TOTAL), lambda i: (i, 0)),
            pl.BlockSpec((blk, N_CONT), lambda i: (i, 0)),
            pl.BlockSpec((EMB_TOTAL, 128), lambda i: (0, 0)),
            pl.BlockSpec((N_CONT, 128), lambda i: (0, 0)),
            pl.BlockSpec((1, 128), lambda i: (0, 0)),
            pl.BlockSpec((128, 64), lambda i: (0, 0)),
            pl.BlockSpec((1, 64), lambda i: (0, 0)),
            pl.BlockSpec((64, OUT), lambda i: (0, 0)),
            pl.BlockSpec((1, OUT), lambda i: (0, 0)),
        ],
        out_specs=pl.BlockSpec((blk, OUT), lambda i: (i, 0)),
        out_shape=jax.ShapeDtypeStruct((BATCH, OUT), jnp.float32),
    )(embs, x_cont, w1e, w1c, b1, w2, b2, w3, b3)


def kernel(x_cat, x_cont, tables, W1, b1, W2, b2, W3, b3):
    offsets = jnp.arange(N_FIELDS, dtype=jnp.int32) * VOCAB
    xi = x_cat.astype(jnp.int32)
    perm = (xi % ROWS_PER_FIELD) * 8 + xi // ROWS_PER_FIELD
    gidx = (perm + offsets[None, :]).reshape(1, TOTAL_IDX)
    tabT = jnp.transpose(tables, (0, 2, 1))
    t128 = _tc_transpose(tabT)
    tables_flat = t128.reshape(N_FIELDS * VOCAB, EMB)
    rows = _sc_gather(tables_flat, gidx)
    embs = rows.reshape(BATCH, EMB_TOTAL)
    return _tc_mlp(
        embs, x_cont,
        W1[:EMB_TOTAL], W1[EMB_TOTAL:],
        b1.reshape(1, 128), W2, b2.reshape(1, 64), W3, b3.reshape(1, OUT),
    )


# speedup vs baseline: 4.2720x; 1.4447x over previous
import functools

import jax
import jax.numpy as jnp
from jax.experimental import pallas as pl
from jax.experimental.pallas import tpu as pltpu
from jax.experimental.pallas import tpu_sc as plsc

N_FIELDS = 26
VOCAB = 100000
EMB = 16
N_CONT = 13
BATCH = 16384
OUT = 2
EMB_TOTAL = N_FIELDS * EMB
TOTAL_IDX = BATCH * N_FIELDS
WINDOW = 128
NUM_WINDOWS = TOTAL_IDX // WINDOW

ROWS_PER_FIELD = VOCAB * EMB // 128
ROWS_PER_FIELD_PAD = 12504
FIELD_STRIDE = ROWS_PER_FIELD_PAD * 8


def _transpose_body(in_ref, out_ref):
    x = in_ref[0]
    y = jnp.concatenate(
        [x[:, j * ROWS_PER_FIELD:(j + 1) * ROWS_PER_FIELD] for j in range(8)],
        axis=0)
    out_ref[0, :ROWS_PER_FIELD, :] = y.T


def _tc_transpose(tabT):
    return pl.pallas_call(
        _transpose_body,
        grid=(N_FIELDS,),
        in_specs=[pl.BlockSpec((1, EMB, VOCAB), lambda f: (f, 0, 0))],
        out_specs=pl.BlockSpec((1, ROWS_PER_FIELD_PAD, 128), lambda f: (f, 0, 0)),
        out_shape=jax.ShapeDtypeStruct((N_FIELDS, ROWS_PER_FIELD_PAD, 128),
                                       jnp.float32),
    )(tabT)


@jax.jit
def _sc_gather(tables_flat, gidx2d):
    mesh = plsc.VectorSubcoreMesh(core_axis_name="core", subcore_axis_name="subcore")

    @functools.partial(
        pl.kernel,
        out_type=jax.ShapeDtypeStruct((TOTAL_IDX, EMB), jnp.float32),
        mesh=mesh,
        compiler_params=pltpu.CompilerParams(use_tc_tiling_on_sc=False),
    )
    def k(tab_hbm, idx_hbm, out_hbm):
        def body(i_vmem, o_vmem):
            pltpu.sync_copy(tab_hbm.at[i_vmem.at[0]], o_vmem)

        pltpu.emit_pipeline(
            body,
            grid=(NUM_WINDOWS,),
            in_specs=[pl.BlockSpec((1, WINDOW), lambda i: (0, i))],
            out_specs=[pl.BlockSpec((WINDOW, EMB), lambda i: (i, 0))],
            core_axis_name=("core", "subcore"),
            dimension_semantics=(pltpu.PARALLEL,),
        )(idx_hbm, out_hbm)

    return k(tables_flat, gidx2d)


def _mlp_body(e_ref, c_ref, w1e_ref, w1c_ref, b1_ref, w2_ref, b2_ref,
              w3_ref, b3_ref, o_ref):
    h = jnp.dot(e_ref[...], w1e_ref[...], preferred_element_type=jnp.float32)
    h = h + jnp.dot(c_ref[...], w1c_ref[...], preferred_element_type=jnp.float32)
    h = jnp.maximum(h + b1_ref[...], 0.0)
    h = jnp.dot(h, w2_ref[...], preferred_element_type=jnp.float32) + b2_ref[...]
    h = jnp.maximum(h, 0.0)
    o_ref[...] = jnp.dot(h, w3_ref[...], preferred_element_type=jnp.float32) + b3_ref[...]


def _tc_mlp(embs, x_cont, w1e, w1c, b1, w2, b2, w3, b3):
    blk = 2048
    grid = BATCH // blk
    return pl.pallas_call(
        _mlp_body,
        grid=(grid,),
        in_specs=[
            pl.BlockSpec((blk, EMB_TOTAL), lambda i: (i, 0)),
            pl.BlockSpec((blk, N_CONT), lambda i: (i, 0)),
            pl.BlockSpec((EMB_TOTAL, 128), lambda i: (0, 0)),
            pl.BlockSpec((N_CONT, 128), lambda i: (0, 0)),
            pl.BlockSpec((1, 128), lambda i: (0, 0)),
            pl.BlockSpec((128, 64), lambda i: (0, 0)),
            pl.BlockSpec((1, 64), lambda i: (0, 0)),
            pl.BlockSpec((64, OUT), lambda i: (0, 0)),
            pl.BlockSpec((1, OUT), lambda i: (0, 0)),
        ],
        out_specs=pl.BlockSpec((blk, OUT), lambda i: (i, 0)),
        out_shape=jax.ShapeDtypeStruct((BATCH, OUT), jnp.float32),
    )(embs, x_cont, w1e, w1c, b1, w2, b2, w3, b3)


def kernel(x_cat, x_cont, tables, W1, b1, W2, b2, W3, b3):
    offsets = jnp.arange(N_FIELDS, dtype=jnp.int32) * FIELD_STRIDE
    xi = x_cat.astype(jnp.int32)
    perm = (xi % ROWS_PER_FIELD) * 8 + xi // ROWS_PER_FIELD
    gidx = (perm + offsets[None, :]).reshape(1, TOTAL_IDX)
    tabT = jnp.transpose(tables, (0, 2, 1))
    t128 = _tc_transpose(tabT)
    tables_flat = t128.reshape(N_FIELDS * FIELD_STRIDE, EMB)
    rows = _sc_gather(tables_flat, gidx)
    embs = rows.reshape(BATCH, EMB_TOTAL)
    return _tc_mlp(
        embs, x_cont,
        W1[:EMB_TOTAL], W1[EMB_TOTAL:],
        b1.reshape(1, 128), W2, b2.reshape(1, 64), W3, b3.reshape(1, OUT),
    )


# speedup vs baseline: 4.6297x; 1.0837x over previous
import functools

import jax
import jax.numpy as jnp
from jax.experimental import pallas as pl
from jax.experimental.pallas import tpu as pltpu
from jax.experimental.pallas import tpu_sc as plsc

N_FIELDS = 26
VOCAB = 100000
EMB = 16
N_CONT = 13
BATCH = 16384
OUT = 2
EMB_TOTAL = N_FIELDS * EMB
WINDOW = 128

ROWS_PER_FIELD = VOCAB * EMB // 128
ROWS_PER_FIELD_PAD = 12504
FIELD_STRIDE = ROWS_PER_FIELD_PAD * 8

NF_HALF = N_FIELDS // 2
HALF_IDX = BATCH * NF_HALF
HALF_WINDOWS = HALF_IDX // WINDOW


def _transpose_body(in_ref, out_ref):
    x = in_ref[0]
    y = jnp.concatenate(
        [x[:, j * ROWS_PER_FIELD:(j + 1) * ROWS_PER_FIELD] for j in range(8)],
        axis=0)
    out_ref[0, :ROWS_PER_FIELD, :] = y.T


def _tc_transpose(tabT, f0):
    return pl.pallas_call(
        _transpose_body,
        grid=(NF_HALF,),
        in_specs=[pl.BlockSpec((1, EMB, VOCAB), lambda f: (f + f0, 0, 0))],
        out_specs=pl.BlockSpec((1, ROWS_PER_FIELD_PAD, 128), lambda f: (f, 0, 0)),
        out_shape=jax.ShapeDtypeStruct((NF_HALF, ROWS_PER_FIELD_PAD, 128),
                                       jnp.float32),
    )(tabT)


@jax.jit
def _sc_gather(tables_flat, gidx2d):
    mesh = plsc.VectorSubcoreMesh(core_axis_name="core", subcore_axis_name="subcore")

    @functools.partial(
        pl.kernel,
        out_type=jax.ShapeDtypeStruct((HALF_IDX, EMB), jnp.float32),
        mesh=mesh,
        compiler_params=pltpu.CompilerParams(use_tc_tiling_on_sc=False),
    )
    def k(tab_hbm, idx_hbm, out_hbm):
        def body(i_vmem, o_vmem):
            pltpu.sync_copy(tab_hbm.at[i_vmem.at[0]], o_vmem)

        pltpu.emit_pipeline(
            body,
            grid=(HALF_WINDOWS,),
            in_specs=[pl.BlockSpec((1, WINDOW), lambda i: (0, i))],
            out_specs=[pl.BlockSpec((WINDOW, EMB), lambda i: (i, 0))],
            core_axis_name=("core", "subcore"),
            dimension_semantics=(pltpu.PARALLEL,),
        )(idx_hbm, out_hbm)

    return k(tables_flat, gidx2d)


def _mlp_body(ea_ref, eb_ref, c_ref, w1a_ref, w1b_ref, w1c_ref, b1_ref,
              w2_ref, b2_ref, w3_ref, b3_ref, o_ref):
    h = jnp.dot(ea_ref[...], w1a_ref[...], preferred_element_type=jnp.float32)
    h = h + jnp.dot(eb_ref[...], w1b_ref[...], preferred_element_type=jnp.float32)
    h = h + jnp.dot(c_ref[...], w1c_ref[...], preferred_element_type=jnp.float32)
    h = jnp.maximum(h + b1_ref[...], 0.0)
    h = jnp.dot(h, w2_ref[...], preferred_element_type=jnp.float32) + b2_ref[...]
    h = jnp.maximum(h, 0.0)
    o_ref[...] = jnp.dot(h, w3_ref[...], preferred_element_type=jnp.float32) + b3_ref[...]


def _tc_mlp(ea, eb, x_cont, w1a, w1b, w1c, b1, w2, b2, w3, b3):
    blk = 2048
    grid = BATCH // blk
    ehalf = NF_HALF * EMB
    return pl.pallas_call(
        _mlp_body,
        grid=(grid,),
        in_specs=[
            pl.BlockSpec((blk, ehalf), lambda i: (i, 0)),
            pl.BlockSpec((blk, ehalf), lambda i: (i, 0)),
            pl.BlockSpec((blk, N_CONT), lambda i: (i, 0)),
            pl.BlockSpec((ehalf, 128), lambda i: (0, 0)),
            pl.BlockSpec((ehalf, 128), lambda i: (0, 0)),
            pl.BlockSpec((N_CONT, 128), lambda i: (0, 0)),
            pl.BlockSpec((1, 128), lambda i: (0, 0)),
            pl.BlockSpec((128, 64), lambda i: (0, 0)),
            pl.BlockSpec((1, 64), lambda i: (0, 0)),
            pl.BlockSpec((64, OUT), lambda i: (0, 0)),
            pl.BlockSpec((1, OUT), lambda i: (0, 0)),
        ],
        out_specs=pl.BlockSpec((blk, OUT), lambda i: (i, 0)),
        out_shape=jax.ShapeDtypeStruct((BATCH, OUT), jnp.float32),
    )(ea, eb, x_cont, w1a, w1b, w1c, b1, w2, b2, w3, b3)


def kernel(x_cat, x_cont, tables, W1, b1, W2, b2, W3, b3):
    xi = x_cat.astype(jnp.int32)
    perm = (xi % ROWS_PER_FIELD) * 8 + xi // ROWS_PER_FIELD
    loffs = jnp.arange(NF_HALF, dtype=jnp.int32) * FIELD_STRIDE
    gidx_a = (perm[:, :NF_HALF] + loffs[None, :]).reshape(1, HALF_IDX)
    gidx_b = (perm[:, NF_HALF:] + loffs[None, :]).reshape(1, HALF_IDX)

    tabT = jnp.transpose(tables, (0, 2, 1))
    t_a = _tc_transpose(tabT, 0)
    rows_a = _sc_gather(t_a.reshape(NF_HALF * FIELD_STRIDE, EMB), gidx_a)
    t_b = _tc_transpose(tabT, NF_HALF)
    rows_b = _sc_gather(t_b.reshape(NF_HALF * FIELD_STRIDE, EMB), gidx_b)

    ea = rows_a.reshape(BATCH, NF_HALF * EMB)
    eb = rows_b.reshape(BATCH, NF_HALF * EMB)
    return _tc_mlp(
        ea, eb, x_cont,
        W1[:NF_HALF * EMB], W1[NF_HALF * EMB:EMB_TOTAL], W1[EMB_TOTAL:],
        b1.reshape(1, 128), W2, b2.reshape(1, 64), W3, b3.reshape(1, OUT),
    )


# speedup vs baseline: 5.0692x; 1.0949x over previous
import functools

import jax
import jax.numpy as jnp
from jax.experimental import pallas as pl
from jax.experimental.pallas import tpu as pltpu
from jax.experimental.pallas import tpu_sc as plsc

N_FIELDS = 26
VOCAB = 100000
EMB = 16
N_CONT = 13
BATCH = 16384
OUT = 2
EMB_TOTAL = N_FIELDS * EMB
WINDOW = 128

ROWS_PER_FIELD = VOCAB * EMB // 128
ROWS_PER_FIELD_PAD = 12504
FIELD_STRIDE = ROWS_PER_FIELD_PAD * 8

NF_HALF = N_FIELDS // 2
HALF_IDX = BATCH * NF_HALF
HALF_WINDOWS = HALF_IDX // WINDOW


def _transpose_body(in_ref, out_ref):
    x = in_ref[0]
    y = jnp.concatenate(
        [x[:, j * ROWS_PER_FIELD:(j + 1) * ROWS_PER_FIELD] for j in range(8)],
        axis=0)
    out_ref[0, :ROWS_PER_FIELD, :] = y.T


def _tc_transpose(tabT, f0):
    return pl.pallas_call(
        _transpose_body,
        grid=(NF_HALF,),
        in_specs=[pl.BlockSpec((1, EMB, VOCAB), lambda f: (f + f0, 0, 0))],
        out_specs=pl.BlockSpec((1, ROWS_PER_FIELD_PAD, 128), lambda f: (f, 0, 0)),
        out_shape=jax.ShapeDtypeStruct((NF_HALF, ROWS_PER_FIELD_PAD, 128),
                                       jnp.float32),
    )(tabT)


N_WORKERS = 32
W_PER_TILE = HALF_WINDOWS // N_WORKERS
IDX_PER_TILE = W_PER_TILE * WINDOW


@jax.jit
def _sc_gather(tables_flat, gidx2d):
    from jax import lax
    mesh = plsc.VectorSubcoreMesh(core_axis_name="core", subcore_axis_name="subcore")

    @functools.partial(
        pl.kernel,
        out_type=jax.ShapeDtypeStruct((HALF_IDX, EMB), jnp.float32),
        mesh=mesh,
        scratch_types=[
            pltpu.VMEM((W_PER_TILE, WINDOW), jnp.int32),
            pltpu.VMEM((IDX_PER_TILE, EMB), jnp.float32),
            pltpu.SemaphoreType.DMA,
        ],
        compiler_params=pltpu.CompilerParams(use_tc_tiling_on_sc=False),
    )
    def k(tab_hbm, idx_hbm, out_hbm, idx_v, rows_v, sem):
        wid = lax.axis_index("subcore") * 2 + lax.axis_index("core")
        pltpu.sync_copy(idx_hbm.at[pl.ds(wid * W_PER_TILE, W_PER_TILE)], idx_v)

        @pl.loop(0, W_PER_TILE)
        def _fire(j):
            pltpu.async_copy(tab_hbm.at[idx_v.at[j]],
                             rows_v.at[pl.ds(j * WINDOW, WINDOW)], sem)

        @pl.loop(0, W_PER_TILE)
        def _drain(j):
            pltpu.make_async_copy(tab_hbm.at[idx_v.at[j]],
                                  rows_v.at[pl.ds(j * WINDOW, WINDOW)], sem).wait()

        pltpu.sync_copy(rows_v, out_hbm.at[pl.ds(wid * IDX_PER_TILE, IDX_PER_TILE)])

    return k(tables_flat, gidx2d)


def _mlp_body(ea_ref, eb_ref, c_ref, w1a_ref, w1b_ref, w1c_ref, b1_ref,
              w2_ref, b2_ref, w3_ref, b3_ref, o_ref):
    h = jnp.dot(ea_ref[...], w1a_ref[...], preferred_element_type=jnp.float32)
    h = h + jnp.dot(eb_ref[...], w1b_ref[...], preferred_element_type=jnp.float32)
    h = h + jnp.dot(c_ref[...], w1c_ref[...], preferred_element_type=jnp.float32)
    h = jnp.maximum(h + b1_ref[...], 0.0)
    h = jnp.dot(h, w2_ref[...], preferred_element_type=jnp.float32) + b2_ref[...]
    h = jnp.maximum(h, 0.0)
    o_ref[...] = jnp.dot(h, w3_ref[...], preferred_element_type=jnp.float32) + b3_ref[...]


def _tc_mlp(ea, eb, x_cont, w1a, w1b, w1c, b1, w2, b2, w3, b3):
    blk = 2048
    grid = BATCH // blk
    ehalf = NF_HALF * EMB
    return pl.pallas_call(
        _mlp_body,
        grid=(grid,),
        in_specs=[
            pl.BlockSpec((blk, ehalf), lambda i: (i, 0)),
            pl.BlockSpec((blk, ehalf), lambda i: (i, 0)),
            pl.BlockSpec((blk, N_CONT), lambda i: (i, 0)),
            pl.BlockSpec((ehalf, 128), lambda i: (0, 0)),
            pl.BlockSpec((ehalf, 128), lambda i: (0, 0)),
            pl.BlockSpec((N_CONT, 128), lambda i: (0, 0)),
            pl.BlockSpec((1, 128), lambda i: (0, 0)),
            pl.BlockSpec((128, 64), lambda i: (0, 0)),
            pl.BlockSpec((1, 64), lambda i: (0, 0)),
            pl.BlockSpec((64, OUT), lambda i: (0, 0)),
            pl.BlockSpec((1, OUT), lambda i: (0, 0)),
        ],
        out_specs=pl.BlockSpec((blk, OUT), lambda i: (i, 0)),
        out_shape=jax.ShapeDtypeStruct((BATCH, OUT), jnp.float32),
    )(ea, eb, x_cont, w1a, w1b, w1c, b1, w2, b2, w3, b3)


def kernel(x_cat, x_cont, tables, W1, b1, W2, b2, W3, b3):
    xi = x_cat.astype(jnp.int32)
    perm = (xi % ROWS_PER_FIELD) * 8 + xi // ROWS_PER_FIELD
    loffs = jnp.arange(NF_HALF, dtype=jnp.int32) * FIELD_STRIDE
    gidx_a = (perm[:, :NF_HALF] + loffs[None, :]).reshape(HALF_WINDOWS, WINDOW)
    gidx_b = (perm[:, NF_HALF:] + loffs[None, :]).reshape(HALF_WINDOWS, WINDOW)

    tabT = jnp.transpose(tables, (0, 2, 1))
    t_a = _tc_transpose(tabT, 0)
    rows_a = _sc_gather(t_a.reshape(NF_HALF * FIELD_STRIDE, EMB), gidx_a)
    t_b = _tc_transpose(tabT, NF_HALF)
    rows_b = _sc_gather(t_b.reshape(NF_HALF * FIELD_STRIDE, EMB), gidx_b)

    ea = rows_a.reshape(BATCH, NF_HALF * EMB)
    eb = rows_b.reshape(BATCH, NF_HALF * EMB)
    return _tc_mlp(
        ea, eb, x_cont,
        W1[:NF_HALF * EMB], W1[NF_HALF * EMB:EMB_TOTAL], W1[EMB_TOTAL:],
        b1.reshape(1, 128), W2, b2.reshape(1, 64), W3, b3.reshape(1, OUT),
    )


# speedup vs baseline: 5.4016x; 1.0656x over previous
import functools

import jax
import jax.numpy as jnp
from jax import lax
from jax.experimental import pallas as pl
from jax.experimental.pallas import tpu as pltpu
from jax.experimental.pallas import tpu_sc as plsc

N_FIELDS = 26
VOCAB = 100000
EMB = 16
N_CONT = 13
BATCH = 16384
OUT = 2
EMB_TOTAL = N_FIELDS * EMB
WINDOW = 128

ROWS_PER_FIELD = VOCAB * EMB // 128
ROWS_PER_FIELD_PAD = 12504
FIELD_STRIDE = ROWS_PER_FIELD_PAD * 8

NF_HALF = N_FIELDS // 2
EMB_HALF = NF_HALF * EMB
HALF_IDX = BATCH * NF_HALF
HALF_WINDOWS = HALF_IDX // WINDOW
WINDOWS_PER_FIELD = BATCH // WINDOW

N_WORKERS = 32
W_PER_TILE = HALF_WINDOWS // N_WORKERS
IDX_PER_TILE = W_PER_TILE * WINDOW


def _transpose_body(in_ref, i_ref, out_ref):
    x = in_ref[0]
    y = jnp.concatenate(
        [x[:, j * ROWS_PER_FIELD:(j + 1) * ROWS_PER_FIELD] for j in range(8)],
        axis=0)
    out_ref[0, :ROWS_PER_FIELD, :] = jax.lax.dot_general(
        y, i_ref[...], (((0,), (0,)), ((), ())),
        preferred_element_type=jnp.float32)


def _tc_transpose(tabT, ident, f0):
    return pl.pallas_call(
        _transpose_body,
        grid=(NF_HALF,),
        in_specs=[
            pl.BlockSpec((1, EMB, VOCAB), lambda f: (f + f0, 0, 0)),
            pl.BlockSpec((128, 128), lambda f: (0, 0)),
        ],
        out_specs=pl.BlockSpec((1, ROWS_PER_FIELD_PAD, 128), lambda f: (f, 0, 0)),
        out_shape=jax.ShapeDtypeStruct((NF_HALF, ROWS_PER_FIELD_PAD, 128),
                                       jnp.float32),
    )(tabT, ident)


@jax.jit
def _sc_gather(tables_flat, gidx2d):
    mesh = plsc.VectorSubcoreMesh(core_axis_name="core", subcore_axis_name="subcore")

    @functools.partial(
        pl.kernel,
        out_type=jax.ShapeDtypeStruct((BATCH, EMB_HALF), jnp.float32),
        mesh=mesh,
        scratch_types=[
            pltpu.VMEM((W_PER_TILE, WINDOW), jnp.int32),
            pltpu.VMEM((IDX_PER_TILE, EMB), jnp.float32),
            pltpu.SemaphoreType.DMA,
            pltpu.SemaphoreType.DMA,
        ],
        compiler_params=pltpu.CompilerParams(use_tc_tiling_on_sc=False),
    )
    def k(tab_hbm, idx_hbm, out_hbm, idx_v, rows_v, gsem, wsem):
        wid = lax.axis_index("subcore") * 2 + lax.axis_index("core")
        pltpu.sync_copy(idx_hbm.at[pl.ds(wid * W_PER_TILE, W_PER_TILE)], idx_v)

        @pl.loop(0, W_PER_TILE)
        def _fire(j):
            pltpu.async_copy(tab_hbm.at[idx_v.at[j]],
                             rows_v.at[pl.ds(j * WINDOW, WINDOW)], gsem)

        @pl.loop(0, W_PER_TILE)
        def _drain(j):
            pltpu.make_async_copy(tab_hbm.at[idx_v.at[j]],
                                  rows_v.at[pl.ds(j * WINDOW, WINDOW)], gsem).wait()
            w = wid * W_PER_TILE + j
            f = w // WINDOWS_PER_FIELD
            kk = w % WINDOWS_PER_FIELD
            pltpu.async_copy(
                rows_v.at[pl.ds(j * WINDOW, WINDOW)],
                out_hbm.at[pl.ds(kk * WINDOW, WINDOW), pl.ds(f * EMB, EMB)],
                wsem)

        @pl.loop(0, W_PER_TILE)
        def _drain_writes(j):
            w = wid * W_PER_TILE + j
            f = w // WINDOWS_PER_FIELD
            kk = w % WINDOWS_PER_FIELD
            pltpu.make_async_copy(
                rows_v.at[pl.ds(j * WINDOW, WINDOW)],
                out_hbm.at[pl.ds(kk * WINDOW, WINDOW), pl.ds(f * EMB, EMB)],
                wsem).wait()

    return k(tables_flat, gidx2d)


def _mlp_body(ea_ref, eb_ref, c_ref, w1a_ref, w1b_ref, w1c_ref, b1_ref,
              w2_ref, b2_ref, w3_ref, b3_ref, o_ref):
    h = jnp.dot(ea_ref[...], w1a_ref[...], preferred_element_type=jnp.float32)
    h = h + jnp.dot(eb_ref[...], w1b_ref[...], preferred_element_type=jnp.float32)
    h = h + jnp.dot(c_ref[...], w1c_ref[...], preferred_element_type=jnp.float32)
    h = jnp.maximum(h + b1_ref[...], 0.0)
    h = jnp.dot(h, w2_ref[...], preferred_element_type=jnp.float32) + b2_ref[...]
    h = jnp.maximum(h, 0.0)
    o_ref[...] = jnp.dot(h, w3_ref[...], preferred_element_type=jnp.float32) + b3_ref[...]


def _tc_mlp(ea, eb, x_cont, w1a, w1b, w1c, b1, w2, b2, w3, b3):
    blk = 2048
    grid = BATCH // blk
    return pl.pallas_call(
        _mlp_body,
        grid=(grid,),
        in_specs=[
            pl.BlockSpec((blk, EMB_HALF), lambda i: (i, 0)),
            pl.BlockSpec((blk, EMB_HALF), lambda i: (i, 0)),
            pl.BlockSpec((blk, N_CONT), lambda i: (i, 0)),
            pl.BlockSpec((EMB_HALF, 128), lambda i: (0, 0)),
            pl.BlockSpec((EMB_HALF, 128), lambda i: (0, 0)),
            pl.BlockSpec((N_CONT, 128), lambda i: (0, 0)),
            pl.BlockSpec((1, 128), lambda i: (0, 0)),
            pl.BlockSpec((128, 64), lambda i: (0, 0)),
            pl.BlockSpec((1, 64), lambda i: (0, 0)),
            pl.BlockSpec((64, OUT), lambda i: (0, 0)),
            pl.BlockSpec((1, OUT), lambda i: (0, 0)),
        ],
        out_specs=pl.BlockSpec((blk, OUT), lambda i: (i, 0)),
        out_shape=jax.ShapeDtypeStruct((BATCH, OUT), jnp.float32),
    )(ea, eb, x_cont, w1a, w1b, w1c, b1, w2, b2, w3, b3)


def kernel(x_cat, x_cont, tables, W1, b1, W2, b2, W3, b3):
    xiT = x_cat.T.astype(jnp.int32)
    permT = (xiT % ROWS_PER_FIELD) * 8 + xiT // ROWS_PER_FIELD
    loffs = jnp.arange(NF_HALF, dtype=jnp.int32) * FIELD_STRIDE
    gidx_a = (permT[:NF_HALF] + loffs[:, None]).reshape(HALF_WINDOWS, WINDOW)
    gidx_b = (permT[NF_HALF:] + loffs[:, None]).reshape(HALF_WINDOWS, WINDOW)

    ident = jnp.eye(128, dtype=jnp.float32)
    tabT = jnp.transpose(tables, (0, 2, 1))
    t_a = _tc_transpose(tabT, ident, 0)
    ea = _sc_gather(t_a.reshape(NF_HALF * FIELD_STRIDE, EMB), gidx_a)
    t_b = _tc_transpose(tabT, ident, NF_HALF)
    eb = _sc_gather(t_b.reshape(NF_HALF * FIELD_STRIDE, EMB), gidx_b)

    return _tc_mlp(
        ea, eb, x_cont,
        W1[:EMB_HALF], W1[EMB_HALF:EMB_TOTAL], W1[EMB_TOTAL:],
        b1.reshape(1, 128), W2, b2.reshape(1, 64), W3, b3.reshape(1, OUT),
    )


# speedup vs baseline: 6.3714x; 1.1795x over previous
import functools

import jax
import jax.numpy as jnp
from jax import lax
from jax.experimental import pallas as pl
from jax.experimental.pallas import tpu as pltpu
from jax.experimental.pallas import tpu_sc as plsc

N_FIELDS = 26
VOCAB = 100000
EMB = 16
N_CONT = 13
BATCH = 16384
OUT = 2
EMB_TOTAL = N_FIELDS * EMB
WINDOW = 128

ROWS_PER_FIELD = VOCAB * EMB // 128
ROWS_PER_FIELD_PAD = 12504
FIELD_STRIDE = ROWS_PER_FIELD_PAD * 8

CHUNKS = (8, 8, 8, 2)
WINDOWS_PER_FIELD = BATCH // WINDOW
N_WORKERS = 32


def _transpose_body(in_ref, i_ref, out_ref):
    x = in_ref[0]
    y = jnp.concatenate(
        [x[:, j * ROWS_PER_FIELD:(j + 1) * ROWS_PER_FIELD] for j in range(8)],
        axis=0)
    out_ref[0, :ROWS_PER_FIELD, :] = jax.lax.dot_general(
        y, i_ref[...], (((0,), (0,)), ((), ())),
        preferred_element_type=jnp.float32)


def _tc_transpose(tabT, ident, f0, nf):
    return pl.pallas_call(
        _transpose_body,
        grid=(nf,),
        in_specs=[
            pl.BlockSpec((1, EMB, VOCAB), lambda f: (f + f0, 0, 0)),
            pl.BlockSpec((128, 128), lambda f: (0, 0)),
        ],
        out_specs=pl.BlockSpec((1, ROWS_PER_FIELD_PAD, 128), lambda f: (f, 0, 0)),
        out_shape=jax.ShapeDtypeStruct((nf, ROWS_PER_FIELD_PAD, 128),
                                       jnp.float32),
    )(tabT, ident)


def _sc_gather(tables_flat, gidx2d, nf):
    mesh = plsc.VectorSubcoreMesh(core_axis_name="core", subcore_axis_name="subcore")
    n_windows = nf * WINDOWS_PER_FIELD
    w_per_tile = n_windows // N_WORKERS
    idx_per_tile = w_per_tile * WINDOW

    @functools.partial(
        pl.kernel,
        out_type=jax.ShapeDtypeStruct((BATCH, EMB * nf), jnp.float32),
        mesh=mesh,
        scratch_types=[
            pltpu.VMEM((w_per_tile, WINDOW), jnp.int32),
            pltpu.VMEM((idx_per_tile, EMB), jnp.float32),
            pltpu.SemaphoreType.DMA,
            pltpu.SemaphoreType.DMA,
        ],
        compiler_params=pltpu.CompilerParams(use_tc_tiling_on_sc=False),
    )
    def k(tab_hbm, idx_hbm, out_hbm, idx_v, rows_v, gsem, wsem):
        wid = lax.axis_index("subcore") * 2 + lax.axis_index("core")
        pltpu.sync_copy(idx_hbm.at[pl.ds(wid * w_per_tile, w_per_tile)], idx_v)

        @pl.loop(0, w_per_tile)
        def _fire(j):
            pltpu.async_copy(tab_hbm.at[idx_v.at[j]],
                             rows_v.at[pl.ds(j * WINDOW, WINDOW)], gsem)

        @pl.loop(0, w_per_tile)
        def _drain(j):
            pltpu.make_async_copy(tab_hbm.at[idx_v.at[j]],
                                  rows_v.at[pl.ds(j * WINDOW, WINDOW)], gsem).wait()
            w = wid * w_per_tile + j
            f = w // WINDOWS_PER_FIELD
            kk = w % WINDOWS_PER_FIELD
            pltpu.async_copy(
                rows_v.at[pl.ds(j * WINDOW, WINDOW)],
                out_hbm.at[pl.ds(kk * WINDOW, WINDOW), pl.ds(f * EMB, EMB)],
                wsem)

        @pl.loop(0, w_per_tile)
        def _drain_writes(j):
            w = wid * w_per_tile + j
            f = w // WINDOWS_PER_FIELD
            kk = w % WINDOWS_PER_FIELD
            pltpu.make_async_copy(
                rows_v.at[pl.ds(j * WINDOW, WINDOW)],
                out_hbm.at[pl.ds(kk * WINDOW, WINDOW), pl.ds(f * EMB, EMB)],
                wsem).wait()

    return k(tables_flat, gidx2d)


def _mlp_body(e0_ref, e1_ref, e2_ref, e3_ref, c_ref, w0_ref, w1_ref, w2_ref,
              w3_ref, wc_ref, b1_ref, wh_ref, b2_ref, wo_ref, b3_ref, o_ref):
    h = jnp.dot(e0_ref[...], w0_ref[...], preferred_element_type=jnp.float32)
    h = h + jnp.dot(e1_ref[...], w1_ref[...], preferred_element_type=jnp.float32)
    h = h + jnp.dot(e2_ref[...], w2_ref[...], preferred_element_type=jnp.float32)
    h = h + jnp.dot(e3_ref[...], w3_ref[...], preferred_element_type=jnp.float32)
    h = h + jnp.dot(c_ref[...], wc_ref[...], preferred_element_type=jnp.float32)
    h = jnp.maximum(h + b1_ref[...], 0.0)
    h = jnp.dot(h, wh_ref[...], preferred_element_type=jnp.float32) + b2_ref[...]
    h = jnp.maximum(h, 0.0)
    o = jnp.dot(h, wo_ref[...], preferred_element_type=jnp.float32) + b3_ref[...]
    o_ref[...] = o.T


def _tc_mlp(es, x_cont, w1s, wc, b1, w2, b2, w3, b3):
    blk = 2048
    grid = BATCH // blk
    e_specs = [pl.BlockSpec((blk, EMB * nf), lambda i: (i, 0)) for nf in CHUNKS]
    w_specs = [pl.BlockSpec((EMB * nf, 128), lambda i: (0, 0)) for nf in CHUNKS]
    return pl.pallas_call(
        _mlp_body,
        grid=(grid,),
        in_specs=e_specs + [pl.BlockSpec((blk, N_CONT), lambda i: (i, 0))]
        + w_specs + [
            pl.BlockSpec((N_CONT, 128), lambda i: (0, 0)),
            pl.BlockSpec((1, 128), lambda i: (0, 0)),
            pl.BlockSpec((128, 64), lambda i: (0, 0)),
            pl.BlockSpec((1, 64), lambda i: (0, 0)),
            pl.BlockSpec((64, OUT), lambda i: (0, 0)),
            pl.BlockSpec((1, OUT), lambda i: (0, 0)),
        ],
        out_specs=pl.BlockSpec((OUT, blk), lambda i: (0, i)),
        out_shape=jax.ShapeDtypeStruct((OUT, BATCH), jnp.float32),
    )(*es, x_cont, *w1s, wc, b1, w2, b2, w3, b3)


def kernel(x_cat, x_cont, tables, W1, b1, W2, b2, W3, b3):
    xiT = x_cat.T.astype(jnp.int32)
    permT = (xiT % ROWS_PER_FIELD) * 8 + xiT // ROWS_PER_FIELD

    ident = jnp.eye(128, dtype=jnp.float32)
    tabT = jnp.transpose(tables, (0, 2, 1))

    es, w1s = [], []
    f0 = 0
    for nf in CHUNKS:
        loffs = jnp.arange(nf, dtype=jnp.int32) * FIELD_STRIDE
        gidx = (permT[f0:f0 + nf] + loffs[:, None]).reshape(
            nf * WINDOWS_PER_FIELD, WINDOW)
        t = _tc_transpose(tabT, ident, f0, nf)
        es.append(_sc_gather(t.reshape(nf * FIELD_STRIDE, EMB), gidx, nf))
        w1s.append(W1[f0 * EMB:(f0 + nf) * EMB])
        f0 += nf

    out_t = _tc_mlp(
        es, x_cont, w1s, W1[EMB_TOTAL:],
        b1.reshape(1, 128), W2, b2.reshape(1, 64), W3, b3.reshape(1, OUT),
    )
    return out_t.T
